# baseline (speedup 1.0000x reference)
import jax
import jax.numpy as jnp
from jax import lax
from jax.experimental import pallas as pl
from jax.experimental.pallas import tpu as pltpu
from jax.experimental.pallas import tpu_sc as plsc

N = 10000
D = 256
E = 160000
NT = 32
RPT = 320
NPAD = NT * RPT
TRASH = RPT
ACC_R = RPT + 8
BLK = 80
NBLK_CAP = E // BLK + 2
ECAP = NBLK_CAP * BLK
SG = 6400
SUBG = SG // 16
MACRO = 4
SRC_SENT = N
LANES = 16


def _mesh():
    return plsc.VectorSubcoreMesh(core_axis_name="c", subcore_axis_name="s")


def _tile_id():
    return lax.axis_index("c") * 16 + lax.axis_index("s")


PACK_SH = 9
SENT_PACKED = SRC_SENT * (1 << PACK_SH) + TRASH


def _partition_body(src_h, dst_h, lp_out, cnt_out,
                    sstage, dstage, pbuf, cbuf):
    g = _tile_id()
    lo = g * RPT
    list_base = g * ECAP

    def flush(cur, blk):
        off = list_base + blk * BLK
        pltpu.sync_copy(pbuf.at[pl.ds(0, BLK)], lp_out.at[pl.ds(off, BLK)])
        for k in range(MACRO):
            rp = pbuf[pl.ds(BLK + k * LANES, LANES)]
            pbuf[pl.ds(k * LANES, LANES)] = rp
        return cur - BLK, blk + 1

    def macro_group(j, carry):
        cur, blk = carry
        for u in range(MACRO):
            sl = pl.ds((j * MACRO + u) * LANES, LANES)
            sv = sstage[sl]
            dv = dstage[sl] - lo
            m = (dv >= 0) & (dv < RPT)
            dvc = jnp.clip(dv, jnp.int32(0), jnp.int32(RPT))
            packed = sv * (1 << PACK_SH) + dvc
            key = jnp.where(m, jnp.int32(0), jnp.int32(1))
            _, ps = plsc.sort_key_val(key, packed)
            pbuf[pl.ds(cur, LANES)] = ps
            npc = plsc.all_reduce_population_count(m)
            npc = npc if npc.ndim == 0 else npc[0]
            cur = cur + npc
        return lax.cond(cur >= BLK, flush, lambda c, b: (c, b), cur, blk)

    def step(o, carry):
        pltpu.sync_copy(src_h.at[pl.ds(o * SG, SG)], sstage)
        pltpu.sync_copy(dst_h.at[pl.ds(o * SG, SG)], dstage)
        return lax.fori_loop(0, SUBG // MACRO, macro_group, carry)

    cur, blk = lax.fori_loop(0, E // SG, step, (jnp.int32(0), jnp.int32(0)))

    sent = jnp.int32(SENT_PACKED) + (lax.iota(jnp.int32, LANES) & 7)
    for k in range(BLK // LANES):
        tl = pl.ds(cur + k * LANES, LANES)
        pbuf[tl] = sent
    _, blk = flush(cur, blk)

    def pad_even(b):
        for k in range(BLK // LANES):
            pbuf[pl.ds(k * LANES, LANES)] = sent
        _, b2 = flush(jnp.int32(BLK), b)
        return b2

    blk = lax.cond(blk % 2 == 1, pad_even, lambda b: b, blk)

    cbuf[pl.ds(0, LANES)] = jnp.full((LANES,), 0, jnp.int32) + blk
    pltpu.sync_copy(cbuf, cnt_out.at[pl.ds(g * LANES, LANES)])


def _layer_body(h_hbm, lp_h, cnt_h, zero_h, out_hbm,
                sidx0, sidx1, didx0, didx1, pblk, chunk0, chunk1,
                accf, cbuf, sem0, sem1):
    g = _tile_id()
    list_base = g * ECAP
    sidx = (sidx0, sidx1)
    didx = (didx0, didx1)
    chunk = (chunk0, chunk1)
    sem = (sem0, sem1)

    pltpu.sync_copy(zero_h, accf)
    pltpu.sync_copy(cnt_h.at[pl.ds(g * LANES, LANES)], cbuf)
    nblk = cbuf[pl.ds(0, LANES)][0]

    def fetch(b, p):
        off = list_base + b * BLK
        pltpu.sync_copy(lp_h.at[pl.ds(off, BLK)], pblk)
        for j in range(BLK // LANES):
            sl = pl.ds(j * LANES, LANES)
            pv = pblk[sl]
            sidx[p][sl] = lax.shift_right_logical(pv, PACK_SH)
            didx[p][sl] = lax.bitwise_and(pv, jnp.int32((1 << PACK_SH) - 1))
        pltpu.async_copy(h_hbm.at[sidx[p]], chunk[p], sem[p])

    def wait(p):
        pltpu.make_async_copy(
            h_hbm.at[pl.ds(0, BLK)], chunk[p], sem[p]).wait()

    def accumulate(p):
        @plsc.parallel_loop(0, BLK // LANES, unroll=2)
        def group(j):
            dv = didx[p][pl.ds(j * LANES, LANES)]
            ebase = j * LANES
            for l in range(LANES):
                arow = dv[l]
                crow = ebase + l
                vals = [chunk[p][crow, pl.ds(c * LANES, LANES)]
                        for c in range(D // LANES)]
                for c in range(D // LANES):
                    plsc.addupdate(accf.at[arow, pl.ds(c * LANES, LANES)],
                                   vals[c])

    fetch(jnp.int32(0), 0)
    fetch(jnp.int32(1), 1)

    def pair(i, carry):
        b = i * 2

        def prefetch(p, nb):
            lax.cond(nb < nblk, lambda: fetch(nb, p), lambda: None)

        wait(0)
        accumulate(0)
        prefetch(0, b + 2)
        wait(1)
        accumulate(1)
        prefetch(1, b + 3)
        return carry

    lax.fori_loop(0, nblk // 2, pair, 0)

    pltpu.sync_copy(accf.at[pl.ds(0, RPT)], out_hbm.at[pl.ds(g * RPT, RPT)])


def _make_partition():
    return pl.kernel(
        _partition_body,
        out_type=(
            jax.ShapeDtypeStruct((NT * ECAP,), jnp.int32),
            jax.ShapeDtypeStruct((NT * LANES,), jnp.int32),
        ),
        mesh=_mesh(),
        scratch_types=[
            pltpu.VMEM((SG,), jnp.int32),
            pltpu.VMEM((SG,), jnp.int32),
            pltpu.VMEM((2 * BLK,), jnp.int32),
            pltpu.VMEM((LANES,), jnp.int32),
        ],
        compiler_params=pltpu.CompilerParams(needs_layout_passes=False),
    )


def _make_layer():
    return pl.kernel(
        _layer_body,
        out_type=jax.ShapeDtypeStruct((NPAD, D), jnp.float32),
        mesh=_mesh(),
        scratch_types=[
            pltpu.VMEM((BLK,), jnp.int32),
            pltpu.VMEM((BLK,), jnp.int32),
            pltpu.VMEM((BLK,), jnp.int32),
            pltpu.VMEM((BLK,), jnp.int32),
            pltpu.VMEM((BLK,), jnp.int32),
            pltpu.VMEM((BLK, D), jnp.float32),
            pltpu.VMEM((BLK, D), jnp.float32),
            pltpu.VMEM((ACC_R, D), jnp.float32),
            pltpu.VMEM((LANES,), jnp.int32),
            pltpu.SemaphoreType.DMA,
            pltpu.SemaphoreType.DMA,
        ],
    )


@jax.jit
def _gin3(hp, src, dst, zrows):
    lp, cnt = _make_partition()(src, dst)
    layer = _make_layer()
    h = hp
    for _ in range(3):
        h = layer(h, lp, cnt, zrows)
    return h[:N]


def kernel(x, edge_index, edge_attr):
    src = edge_index[0].astype(jnp.int32)
    dst = edge_index[1].astype(jnp.int32)
    hp = jnp.concatenate(
        [x, jnp.zeros((NPAD - N, D), jnp.float32)], axis=0)
    zrows = jnp.zeros((ACC_R, D), jnp.float32)
    return _gin3(hp, src, dst, zrows)


# speedup vs baseline: 1.3015x; 1.3015x over previous
import jax
import jax.numpy as jnp
from jax import lax
from jax.experimental import pallas as pl
from jax.experimental.pallas import tpu as pltpu
from jax.experimental.pallas import tpu_sc as plsc

N = 10000
D = 256
E = 160000
NT = 32
RPT = 320
NPAD = NT * RPT
TRASH = RPT
ACC_R = RPT + 8
BLK = 80
NBLK_CAP = E // BLK + 2
ECAP = NBLK_CAP * BLK
SG = 6400
SUBG = SG // 16
MACRO = 4
SRC_SENT = N
LANES = 16


def _mesh():
    return plsc.VectorSubcoreMesh(core_axis_name="c", subcore_axis_name="s")


def _tile_id():
    return lax.axis_index("c") * 16 + lax.axis_index("s")


PACK_SH = 9
SENT_PACKED = SRC_SENT * (1 << PACK_SH) + TRASH


def _partition_body(src_h, dst_h, lp_out, cnt_out,
                    sstage, dstage, pbuf, cbuf):
    g = _tile_id()
    lo = g * RPT
    list_base = g * ECAP

    def flush(cur, blk):
        off = list_base + blk * BLK
        pltpu.sync_copy(pbuf.at[pl.ds(0, BLK)], lp_out.at[pl.ds(off, BLK)])
        for k in range(MACRO):
            rp = pbuf[pl.ds(BLK + k * LANES, LANES)]
            pbuf[pl.ds(k * LANES, LANES)] = rp
        return cur - BLK, blk + 1

    def macro_group(j, carry):
        cur, blk = carry
        for u in range(MACRO):
            sl = pl.ds((j * MACRO + u) * LANES, LANES)
            sv = sstage[sl]
            dv = dstage[sl] - lo
            m = (dv >= 0) & (dv < RPT)
            dvc = jnp.clip(dv, jnp.int32(0), jnp.int32(RPT))
            packed = sv * (1 << PACK_SH) + dvc
            key = jnp.where(m, jnp.int32(0), jnp.int32(1))
            _, ps = plsc.sort_key_val(key, packed)
            pbuf[pl.ds(cur, LANES)] = ps
            npc = plsc.all_reduce_population_count(m)
            npc = npc if npc.ndim == 0 else npc[0]
            cur = cur + npc
        return lax.cond(cur >= BLK, flush, lambda c, b: (c, b), cur, blk)

    def step(o, carry):
        pltpu.sync_copy(src_h.at[pl.ds(o * SG, SG)], sstage)
        pltpu.sync_copy(dst_h.at[pl.ds(o * SG, SG)], dstage)
        return lax.fori_loop(0, SUBG // MACRO, macro_group, carry)

    cur, blk = lax.fori_loop(0, E // SG, step, (jnp.int32(0), jnp.int32(0)))

    sent = jnp.int32(SENT_PACKED) + (lax.iota(jnp.int32, LANES) & 7)
    for k in range(BLK // LANES):
        tl = pl.ds(cur + k * LANES, LANES)
        pbuf[tl] = sent
    _, blk = flush(cur, blk)

    def pad_even(b):
        for k in range(BLK // LANES):
            pbuf[pl.ds(k * LANES, LANES)] = sent
        _, b2 = flush(jnp.int32(BLK), b)
        return b2

    blk = lax.cond(blk % 2 == 1, pad_even, lambda b: b, blk)

    cbuf[pl.ds(0, LANES)] = jnp.full((LANES,), 0, jnp.int32) + blk
    pltpu.sync_copy(cbuf, cnt_out.at[pl.ds(g * LANES, LANES)])


def _layer_body(h_hbm, lp_h, cnt_h, zero_h, out_hbm,
                sidx0, sidx1, didx0, didx1, pblk0, pblk1, chunk0, chunk1,
                accf, cbuf, sem0, sem1, seml0, seml1):
    g = _tile_id()
    list_base = g * ECAP
    sidx = (sidx0, sidx1)
    didx = (didx0, didx1)
    pblk = (pblk0, pblk1)
    chunk = (chunk0, chunk1)
    sem = (sem0, sem1)
    seml = (seml0, seml1)

    pltpu.sync_copy(zero_h, accf)
    pltpu.sync_copy(cnt_h.at[pl.ds(g * LANES, LANES)], cbuf)
    nblk = cbuf[pl.ds(0, LANES)][0]

    def fetch_list(b, p):
        off = list_base + b * BLK
        pltpu.async_copy(lp_h.at[pl.ds(off, BLK)], pblk[p], seml[p])

    def launch(p):
        pltpu.make_async_copy(
            lp_h.at[pl.ds(0, BLK)], pblk[p], seml[p]).wait()
        for j in range(BLK // LANES):
            sl = pl.ds(j * LANES, LANES)
            pv = pblk[p][sl]
            sidx[p][sl] = lax.shift_right_logical(pv, PACK_SH)
            didx[p][sl] = lax.bitwise_and(pv, jnp.int32((1 << PACK_SH) - 1))
        pltpu.async_copy(h_hbm.at[sidx[p]], chunk[p], sem[p])

    def wait(p):
        pltpu.make_async_copy(
            h_hbm.at[pl.ds(0, BLK)], chunk[p], sem[p]).wait()

    def accumulate(p):
        def group(j, carry2):
            dv = didx[p][pl.ds(j * LANES, LANES)]
            ebase = j * LANES
            for l in range(LANES):
                arow = dv[l]
                crow = ebase + l
                vals = [chunk[p][crow, pl.ds(c * LANES, LANES)]
                        for c in range(D // LANES)]
                for c in range(D // LANES):
                    plsc.addupdate(accf.at[arow, pl.ds(c * LANES, LANES)],
                                   vals[c])
            return carry2

        lax.fori_loop(0, BLK // LANES, group, 0)

    fetch_list(jnp.int32(0), 0)
    fetch_list(jnp.int32(1), 1)
    launch(0)
    launch(1)

    def pair(i, carry):
        b = i * 2

        def guarded(nb, fn):
            lax.cond(nb < nblk, fn, lambda: None)

        guarded(b + 2, lambda: fetch_list(b + 2, 0))
        wait(0)
        accumulate(0)
        guarded(b + 2, lambda: launch(0))
        guarded(b + 3, lambda: fetch_list(b + 3, 1))
        wait(1)
        accumulate(1)
        guarded(b + 3, lambda: launch(1))
        return carry

    lax.fori_loop(0, nblk // 2, pair, 0)

    pltpu.sync_copy(accf.at[pl.ds(0, RPT)], out_hbm.at[pl.ds(g * RPT, RPT)])


def _make_partition():
    return pl.kernel(
        _partition_body,
        out_type=(
            jax.ShapeDtypeStruct((NT * ECAP,), jnp.int32),
            jax.ShapeDtypeStruct((NT * LANES,), jnp.int32),
        ),
        mesh=_mesh(),
        scratch_types=[
            pltpu.VMEM((SG,), jnp.int32),
            pltpu.VMEM((SG,), jnp.int32),
            pltpu.VMEM((2 * BLK,), jnp.int32),
            pltpu.VMEM((LANES,), jnp.int32),
        ],
        compiler_params=pltpu.CompilerParams(needs_layout_passes=False),
    )


def _make_layer():
    return pl.kernel(
        _layer_body,
        out_type=jax.ShapeDtypeStruct((NPAD, D), jnp.float32),
        mesh=_mesh(),
        scratch_types=[
            pltpu.VMEM((BLK,), jnp.int32),
            pltpu.VMEM((BLK,), jnp.int32),
            pltpu.VMEM((BLK,), jnp.int32),
            pltpu.VMEM((BLK,), jnp.int32),
            pltpu.VMEM((BLK,), jnp.int32),
            pltpu.VMEM((BLK,), jnp.int32),
            pltpu.VMEM((BLK, D), jnp.float32),
            pltpu.VMEM((BLK, D), jnp.float32),
            pltpu.VMEM((ACC_R, D), jnp.float32),
            pltpu.VMEM((LANES,), jnp.int32),
            pltpu.SemaphoreType.DMA,
            pltpu.SemaphoreType.DMA,
            pltpu.SemaphoreType.DMA,
            pltpu.SemaphoreType.DMA,
        ],
    )


@jax.jit
def _gin3(hp, src, dst, zrows):
    lp, cnt = _make_partition()(src, dst)
    layer = _make_layer()
    h = hp
    for _ in range(3):
        h = layer(h, lp, cnt, zrows)
    return h[:N]


def kernel(x, edge_index, edge_attr):
    src = edge_index[0].astype(jnp.int32)
    dst = edge_index[1].astype(jnp.int32)
    hp = jnp.concatenate(
        [x, jnp.zeros((NPAD - N, D), jnp.float32)], axis=0)
    zrows = jnp.zeros((ACC_R, D), jnp.float32)
    return _gin3(hp, src, dst, zrows)


# speedup vs baseline: 1.3636x; 1.0478x over previous
import jax
import jax.numpy as jnp
from jax import lax
from jax.experimental import pallas as pl
from jax.experimental.pallas import tpu as pltpu
from jax.experimental.pallas import tpu_sc as plsc

N = 10000
D = 256
E = 160000
NT = 32
RPT = 320
NPAD = NT * RPT
TRASH = RPT
ACC_R = RPT + 8
BLK = 80
NBLK_CAP = E // BLK + 2
ECAP = NBLK_CAP * BLK
SG = 8000
SUBG = SG // 16
MACRO = 4
SRC_SENT = N
LANES = 16


def _mesh():
    return plsc.VectorSubcoreMesh(core_axis_name="c", subcore_axis_name="s")


def _tile_id():
    return lax.axis_index("c") * 16 + lax.axis_index("s")


PACK_SH = 9
SENT_PACKED = SRC_SENT * (1 << PACK_SH) + TRASH


def _partition_body(src_h, dst_h, lp_out, cnt_out,
                    sstage0, sstage1, dstage0, dstage1, pbuf, cbuf,
                    sems0, sems1):
    g = _tile_id()
    lo = g * RPT
    list_base = g * ECAP
    sstage = (sstage0, sstage1)
    dstage = (dstage0, dstage1)
    sems = (sems0, sems1)

    def stage(o, p):
        pltpu.async_copy(src_h.at[pl.ds(o * SG, SG)], sstage[p], sems[p])
        pltpu.async_copy(dst_h.at[pl.ds(o * SG, SG)], dstage[p], sems[p])

    def stage_wait(p):
        pltpu.make_async_copy(src_h.at[pl.ds(0, SG)], sstage[p],
                              sems[p]).wait()
        pltpu.make_async_copy(dst_h.at[pl.ds(0, SG)], dstage[p],
                              sems[p]).wait()

    def flush(cur, blk):
        off = list_base + blk * BLK
        pltpu.sync_copy(pbuf.at[pl.ds(0, BLK)], lp_out.at[pl.ds(off, BLK)])
        for k in range(MACRO):
            rp = pbuf[pl.ds(BLK + k * LANES, LANES)]
            pbuf[pl.ds(k * LANES, LANES)] = rp
        return cur - BLK, blk + 1

    def make_macro_group(p):
        def macro_group(j, carry):
            cur, blk = carry
            for u in range(MACRO):
                sl = pl.ds((j * MACRO + u) * LANES, LANES)
                sv = sstage[p][sl]
                dv = dstage[p][sl] - lo
                m = (dv >= 0) & (dv < RPT)
                packed = sv * (1 << PACK_SH) + dv
                key = jnp.where(m, jnp.int32(0), jnp.int32(1))
                _, ps = plsc.sort_key_val(key, packed)
                pbuf[pl.ds(cur, LANES)] = ps
                npc = plsc.all_reduce_population_count(m)
                npc = npc if npc.ndim == 0 else npc[0]
                cur = cur + npc
            return lax.cond(cur >= BLK, flush, lambda c, b: (c, b), cur, blk)
        return macro_group

    stage(jnp.int32(0), 0)
    stage(jnp.int32(1), 1)

    def step_pair(i, carry):
        o = i * 2
        for p in range(2):
            stage_wait(p)
            carry = lax.fori_loop(0, SUBG // MACRO, make_macro_group(p),
                                  carry)
            nxt = o + p + 2

            def prefetch():
                stage(nxt, p)

            lax.cond(nxt < E // SG, prefetch, lambda: None)
        return carry

    cur, blk = lax.fori_loop(0, (E // SG) // 2, step_pair,
                             (jnp.int32(0), jnp.int32(0)))

    sent = jnp.int32(SENT_PACKED) + (lax.iota(jnp.int32, LANES) & 7)
    for k in range(BLK // LANES):
        tl = pl.ds(cur + k * LANES, LANES)
        pbuf[tl] = sent
    _, blk = flush(cur, blk)

    def pad_even(b):
        for k in range(BLK // LANES):
            pbuf[pl.ds(k * LANES, LANES)] = sent
        _, b2 = flush(jnp.int32(BLK), b)
        return b2

    blk = lax.cond(blk % 2 == 1, pad_even, lambda b: b, blk)

    cbuf[pl.ds(0, LANES)] = jnp.full((LANES,), 0, jnp.int32) + blk
    pltpu.sync_copy(cbuf, cnt_out.at[pl.ds(g * LANES, LANES)])


def _layer_body(h_hbm, lp_h, cnt_h, zero_h, out_hbm,
                sidx0, sidx1, didx0, didx1, pblk0, pblk1, chunk0, chunk1,
                accf, cbuf, sem0, sem1, seml0, seml1):
    g = _tile_id()
    list_base = g * ECAP
    sidx = (sidx0, sidx1)
    didx = (didx0, didx1)
    pblk = (pblk0, pblk1)
    chunk = (chunk0, chunk1)
    sem = (sem0, sem1)
    seml = (seml0, seml1)

    pltpu.sync_copy(zero_h, accf)
    pltpu.sync_copy(cnt_h.at[pl.ds(g * LANES, LANES)], cbuf)
    nblk = cbuf[pl.ds(0, LANES)][0]

    def fetch_list(b, p):
        off = list_base + b * BLK
        pltpu.async_copy(lp_h.at[pl.ds(off, BLK)], pblk[p], seml[p])

    def launch(p):
        pltpu.make_async_copy(
            lp_h.at[pl.ds(0, BLK)], pblk[p], seml[p]).wait()
        for j in range(BLK // LANES):
            sl = pl.ds(j * LANES, LANES)
            pv = pblk[p][sl]
            sidx[p][sl] = lax.shift_right_logical(pv, PACK_SH)
            didx[p][sl] = lax.bitwise_and(pv, jnp.int32((1 << PACK_SH) - 1))
        pltpu.async_copy(h_hbm.at[sidx[p]], chunk[p], sem[p])

    def wait(p):
        pltpu.make_async_copy(
            h_hbm.at[pl.ds(0, BLK)], chunk[p], sem[p]).wait()

    def accumulate(p):
        def group(j, carry2):
            dv = didx[p][pl.ds(j * LANES, LANES)]
            ebase = j * LANES
            for l in range(LANES):
                arow = dv[l]
                crow = ebase + l
                vals = [chunk[p][crow, pl.ds(c * LANES, LANES)]
                        for c in range(D // LANES)]
                for c in range(D // LANES):
                    plsc.addupdate(accf.at[arow, pl.ds(c * LANES, LANES)],
                                   vals[c])
            return carry2

        lax.fori_loop(0, BLK // LANES, group, 0)

    fetch_list(jnp.int32(0), 0)
    fetch_list(jnp.int32(1), 1)
    launch(0)
    launch(1)

    def pair(i, carry):
        b = i * 2

        def guarded(nb, fn):
            lax.cond(nb < nblk, fn, lambda: None)

        guarded(b + 2, lambda: fetch_list(b + 2, 0))
        wait(0)
        accumulate(0)
        guarded(b + 2, lambda: launch(0))
        guarded(b + 3, lambda: fetch_list(b + 3, 1))
        wait(1)
        accumulate(1)
        guarded(b + 3, lambda: launch(1))
        return carry

    lax.fori_loop(0, nblk // 2, pair, 0)

    pltpu.sync_copy(accf.at[pl.ds(0, RPT)], out_hbm.at[pl.ds(g * RPT, RPT)])


def _make_partition():
    return pl.kernel(
        _partition_body,
        out_type=(
            jax.ShapeDtypeStruct((NT * ECAP,), jnp.int32),
            jax.ShapeDtypeStruct((NT * LANES,), jnp.int32),
        ),
        mesh=_mesh(),
        scratch_types=[
            pltpu.VMEM((SG,), jnp.int32),
            pltpu.VMEM((SG,), jnp.int32),
            pltpu.VMEM((SG,), jnp.int32),
            pltpu.VMEM((SG,), jnp.int32),
            pltpu.VMEM((2 * BLK,), jnp.int32),
            pltpu.VMEM((LANES,), jnp.int32),
            pltpu.SemaphoreType.DMA,
            pltpu.SemaphoreType.DMA,
        ],
        compiler_params=pltpu.CompilerParams(needs_layout_passes=False),
    )


def _make_layer():
    return pl.kernel(
        _layer_body,
        out_type=jax.ShapeDtypeStruct((NPAD, D), jnp.float32),
        mesh=_mesh(),
        scratch_types=[
            pltpu.VMEM((BLK,), jnp.int32),
            pltpu.VMEM((BLK,), jnp.int32),
            pltpu.VMEM((BLK,), jnp.int32),
            pltpu.VMEM((BLK,), jnp.int32),
            pltpu.VMEM((BLK,), jnp.int32),
            pltpu.VMEM((BLK,), jnp.int32),
            pltpu.VMEM((BLK, D), jnp.float32),
            pltpu.VMEM((BLK, D), jnp.float32),
            pltpu.VMEM((ACC_R, D), jnp.float32),
            pltpu.VMEM((LANES,), jnp.int32),
            pltpu.SemaphoreType.DMA,
            pltpu.SemaphoreType.DMA,
            pltpu.SemaphoreType.DMA,
            pltpu.SemaphoreType.DMA,
        ],
    )


@jax.jit
def _gin3(hp, src, dst, zrows):
    lp, cnt = _make_partition()(src, dst)
    layer = _make_layer()
    h = hp
    for _ in range(3):
        h = layer(h, lp, cnt, zrows)
    return h[:N]


def kernel(x, edge_index, edge_attr):
    src = edge_index[0].astype(jnp.int32)
    dst = edge_index[1].astype(jnp.int32)
    hp = jnp.concatenate(
        [x, jnp.zeros((NPAD - N, D), jnp.float32)], axis=0)
    zrows = jnp.zeros((ACC_R, D), jnp.float32)
    return _gin3(hp, src, dst, zrows)
